# baseline (speedup 1.0000x reference)
import jax
import jax.numpy as jnp
from jax import lax
from jax.experimental import pallas as pl
from jax.experimental.pallas import tpu as pltpu
from jax.experimental.pallas import tpu_sc as plsc

VOCAB = 1000000
EMB_DIM = 64
BATCH = 4096
HIST = 50

NC = 2
NS = 16
NW = NC * NS
BPW = BATCH // NW

_mesh = plsc.VectorSubcoreMesh(core_axis_name="c", subcore_axis_name="s")


def _body(idx_hbm, table_hbm, out_hbm, idx_v, g0, g1, t0, t1,
          gsem0, gsem1, osem0, osem1):
    wid = lax.axis_index("s") * NC + lax.axis_index("c")
    b0 = wid * BPW

    pltpu.sync_copy(idx_hbm.at[wid], idx_v)

    gbuf = (g0, g1)
    tbuf = (t0, t1)
    gsems = (gsem0, gsem1)
    osems = (osem0, osem1)

    def start_gather(h, b):
        pltpu.async_copy(table_hbm.at[idx_v.at[h]], gbuf[b], gsems[b])

    def wait_gather(h, b):
        pltpu.make_async_copy(table_hbm.at[idx_v.at[h]], gbuf[b], gsems[b]).wait()

    def start_out(h, b):
        pltpu.async_copy(tbuf[b], out_hbm.at[h, :, pl.ds(b0, BPW)], osems[b])

    def wait_out(h, b):
        pltpu.make_async_copy(tbuf[b], out_hbm.at[h, :, pl.ds(b0, BPW)],
                              osems[b]).wait()

    def transpose(b):
        for bb0 in range(0, BPW, 16):
            bvec = jax.lax.iota(jnp.int32, 16) + bb0
            for c in range(EMB_DIM):
                cvec = jnp.full((16,), c, jnp.int32)
                tbuf[b][c, pl.ds(bb0, 16)] = plsc.load_gather(
                    gbuf[b], [bvec, cvec])

    start_gather(0, 0)

    @pl.loop(0, HIST, step=2)
    def step(h0):
        wait_gather(h0, 0)
        start_gather(h0 + 1, 1)

        @pl.when(h0 >= 2)
        def _():
            wait_out(h0 - 2, 0)

        transpose(0)
        start_out(h0, 0)

        wait_gather(h0 + 1, 1)

        @pl.when(h0 + 2 < HIST)
        def _():
            start_gather(h0 + 2, 0)

        @pl.when(h0 >= 2)
        def _():
            wait_out(h0 - 1, 1)

        transpose(1)
        start_out(h0 + 1, 1)

    wait_out(HIST - 2, 0)
    wait_out(HIST - 1, 1)


_gather = pl.kernel(
    _body,
    out_type=jax.ShapeDtypeStruct((HIST, EMB_DIM, BATCH), jnp.float32),
    mesh=_mesh,
    scratch_types=[
        pltpu.VMEM((HIST, BPW), jnp.int32),
        pltpu.VMEM((BPW, EMB_DIM), jnp.float32),
        pltpu.VMEM((BPW, EMB_DIM), jnp.float32),
        pltpu.VMEM((EMB_DIM, BPW), jnp.float32),
        pltpu.VMEM((EMB_DIM, BPW), jnp.float32),
        pltpu.SemaphoreType.DMA,
        pltpu.SemaphoreType.DMA,
        pltpu.SemaphoreType.DMA,
        pltpu.SemaphoreType.DMA,
    ],
    compiler_params=pltpu.CompilerParams(use_tc_tiling_on_sc=False,
                                         needs_layout_passes=False),
)


def kernel(src_seq, emb_weight):
    idx = src_seq.astype(jnp.int32).reshape(NW, BPW, HIST).transpose(0, 2, 1)
    out = _gather(idx, emb_weight)
    return out.transpose(2, 0, 1)


# speedup vs baseline: 1.3174x; 1.3174x over previous
import jax
import jax.numpy as jnp
from jax import lax
from jax.experimental import pallas as pl
from jax.experimental.pallas import tpu as pltpu
from jax.experimental.pallas import tpu_sc as plsc

VOCAB = 1000000
EMB_DIM = 64
BATCH = 4096
HIST = 50

NC = 2
NS = 16
NW = NC * NS

TOTAL = BATCH * HIST
CHUNK = 256
NCHUNKS = TOTAL // CHUNK
CPW = NCHUNKS // NW

NBUF = 5
AHEAD = NBUF - 1

_mesh = plsc.VectorSubcoreMesh(core_axis_name="c", subcore_axis_name="s")


def _body(idx_hbm, table_hbm, out_hbm, idx_v, rows, gsems, osems):
    wid = lax.axis_index("s") * NC + lax.axis_index("c")
    c0 = wid * CPW

    pltpu.sync_copy(idx_hbm.at[wid], idx_v)

    def start_gather(j, b):
        pltpu.async_copy(table_hbm.at[idx_v.at[j]], rows[b], gsems[b])

    def wait_gather(j, b):
        pltpu.make_async_copy(table_hbm.at[idx_v.at[j]], rows[b], gsems[b]).wait()

    def start_out(j, b):
        pltpu.async_copy(rows[b], out_hbm.at[c0 + j], osems[b])

    def wait_out(j, b):
        pltpu.make_async_copy(rows[b], out_hbm.at[c0 + j], osems[b]).wait()

    for b in range(AHEAD):
        start_gather(b, b)

    @pl.loop(0, CPW, step=NBUF)
    def step(j0):
        for b in range(NBUF):
            j = j0 + b
            jn = j + AHEAD
            bn = (b + AHEAD) % NBUF

            @pl.when(jn < CPW)
            def _():
                if b == 0:
                    @pl.when(j >= 1)
                    def _():
                        wait_out(j - 1, bn)
                else:
                    wait_out(j - 1, bn)
                start_gather(jn, bn)

            wait_gather(j, b)
            start_out(j, b)

    for b in range(NBUF):
        wait_out(CPW - NBUF + b, b)


_gather = pl.kernel(
    _body,
    out_type=jax.ShapeDtypeStruct((NCHUNKS, CHUNK, EMB_DIM), jnp.float32),
    mesh=_mesh,
    scratch_types=[
        pltpu.VMEM((CPW, CHUNK), jnp.int32),
        [pltpu.VMEM((CHUNK, EMB_DIM), jnp.float32) for _ in range(NBUF)],
        [pltpu.SemaphoreType.DMA for _ in range(NBUF)],
        [pltpu.SemaphoreType.DMA for _ in range(NBUF)],
    ],
    compiler_params=pltpu.CompilerParams(use_tc_tiling_on_sc=False),
)


def kernel(src_seq, emb_weight):
    idx = src_seq.astype(jnp.int32).reshape(NW, CPW, CHUNK)
    out = _gather(idx, emb_weight)
    return out.reshape(BATCH, HIST, EMB_DIM)
